# baseline (speedup 1.0000x reference)
import jax
import jax.numpy as jnp
from jax.experimental import pallas as pl
from jax.experimental.pallas import tpu as pltpu

N = 5000
F = 64
NJ = 100
P = 128
R = 200
NB = N // R


def _fwd_kernel(adj_ref, x_ref, cand_ref, maskf_ref,
                w01_ref, b01_ref, w02_ref, b02_ref, g0_ref, be0_ref,
                w11_ref, b11_ref, w12_ref, b12_ref, g1_ref, be1_ref,
                wa1_ref, ba1_ref, wa2_ref, ba2_ref,
                wc1_ref, bc1_ref, wc2_ref, bc2_ref,
                pi_ref, v_ref,
                h_ref, z_ref):
    l = pl.program_id(0)
    i = pl.program_id(1)
    is0 = l == 0

    @pl.when(is0 & (i == 0))
    def _init():
        h_ref[...] = x_ref[...]

    w1 = jnp.where(is0, w01_ref[...], w11_ref[...])
    b1 = jnp.where(is0, b01_ref[...], b11_ref[...])
    w2 = jnp.where(is0, w02_ref[...], w12_ref[...])
    b2 = jnp.where(is0, b02_ref[...], b12_ref[...])

    pooled = jnp.dot(adj_ref[...], h_ref[...],
                     preferred_element_type=jnp.float32)
    t = jnp.maximum(pooled @ w1 + b1, 0.0)
    z_ref[pl.ds(i * R, R), :] = t @ w2 + b2

    @pl.when(i == NB - 1)
    def _bn():
        gamma = jnp.where(is0, g0_ref[...], g1_ref[...])
        beta = jnp.where(is0, be0_ref[...], be1_ref[...])
        z = z_ref[...]
        mu = jnp.mean(z, axis=0, keepdims=True)
        var = jnp.mean((z - mu) ** 2, axis=0, keepdims=True)
        hn = (z - mu) * jax.lax.rsqrt(var + 1e-5) * gamma + beta
        h_ref[...] = jnp.maximum(hn, 0.0)

    @pl.when((l == 1) & (i == NB - 1))
    def _head():
        h = h_ref[...]
        hp = jnp.mean(h, axis=0, keepdims=True)
        node_ids = jax.lax.broadcasted_iota(jnp.int32, (P, N), 1)
        oh = (node_ids == cand_ref[...]).astype(jnp.float32)
        cf = jnp.dot(oh, h, preferred_element_type=jnp.float32)
        concat = jnp.concatenate(
            [cf, jnp.broadcast_to(hp, (P, F))], axis=1)
        a = jnp.tanh(concat @ wa1_ref[...] + ba1_ref[...])
        s = a @ wa2_ref[...] + ba2_ref[...]
        s = jnp.where(maskf_ref[...] > 0.0, -1e30, s)
        e = jnp.exp(s - jnp.max(s))
        pi_ref[...] = e / jnp.sum(e)
        c = jnp.tanh(hp @ wc1_ref[...] + bc1_ref[...])
        v_ref[...] = c @ wc2_ref[...] + bc2_ref[...]


def kernel(adj, x, candidate, mask,
           W0_1, b0_1, W0_2, b0_2, gamma0, beta0,
           W1_1, b1_1, W1_2, b1_2, gamma1, beta1,
           Wa1, ba1, Wa2, ba2, Wc1, bc1, Wc2, bc2):
    f32 = jnp.float32
    cand_col = jnp.full((P, 1), -1, jnp.int32).at[:NJ, 0].set(candidate[0])
    maskf = jnp.ones((P, 1), f32).at[:NJ, 0].set(mask[0].astype(f32))

    row = lambda v: v.reshape(1, -1).astype(f32)
    full = lambda a: pl.BlockSpec(a.shape, lambda l, i: (0,) * a.ndim)

    operands = (adj, x, cand_col, maskf,
                W0_1, row(b0_1), W0_2, row(b0_2), row(gamma0), row(beta0),
                W1_1, row(b1_1), W1_2, row(b1_2), row(gamma1), row(beta1),
                Wa1, row(ba1), Wa2, row(ba2),
                Wc1, row(bc1), Wc2, row(bc2))
    in_specs = [pl.BlockSpec((R, N), lambda l, i: (i, 0))]
    in_specs += [full(a) for a in operands[1:]]

    pi_pad, v = pl.pallas_call(
        _fwd_kernel,
        grid=(2, NB),
        in_specs=in_specs,
        out_specs=[pl.BlockSpec((P, 1), lambda l, i: (0, 0)),
                   pl.BlockSpec((1, 1), lambda l, i: (0, 0))],
        out_shape=[jax.ShapeDtypeStruct((P, 1), f32),
                   jax.ShapeDtypeStruct((1, 1), f32)],
        scratch_shapes=[pltpu.VMEM((N, F), f32),
                        pltpu.VMEM((N, F), f32)],
    )(*operands)
    return pi_pad[:NJ, 0][None, :], v


# speedup vs baseline: 1.2300x; 1.2300x over previous
import jax
import jax.numpy as jnp
from jax.experimental import pallas as pl
from jax.experimental.pallas import tpu as pltpu

N = 5000
F = 64
NJ = 100
P = 128
R = 1000
NB = N // R


def _fwd_kernel(adj_ref, x_ref, cand_ref, maskf_ref,
                w01_ref, b01_ref, w02_ref, b02_ref, g0_ref, be0_ref,
                w11_ref, b11_ref, w12_ref, b12_ref, g1_ref, be1_ref,
                wa1_ref, ba1_ref, wa2_ref, ba2_ref,
                wc1_ref, bc1_ref, wc2_ref, bc2_ref,
                pi_ref, v_ref,
                h_ref, z_ref):
    l = pl.program_id(0)
    i = pl.program_id(1)
    is0 = l == 0

    @pl.when(is0 & (i == 0))
    def _init():
        h_ref[...] = x_ref[...]

    w1 = jnp.where(is0, w01_ref[...], w11_ref[...])
    b1 = jnp.where(is0, b01_ref[...], b11_ref[...])
    w2 = jnp.where(is0, w02_ref[...], w12_ref[...])
    b2 = jnp.where(is0, b02_ref[...], b12_ref[...])

    pooled = jnp.dot(adj_ref[...].astype(jnp.bfloat16),
                     h_ref[...].astype(jnp.bfloat16),
                     preferred_element_type=jnp.float32)
    t = jnp.maximum(pooled @ w1 + b1, 0.0)
    z_ref[pl.ds(i * R, R), :] = t @ w2 + b2

    @pl.when(i == NB - 1)
    def _bn():
        gamma = jnp.where(is0, g0_ref[...], g1_ref[...])
        beta = jnp.where(is0, be0_ref[...], be1_ref[...])
        z = z_ref[...]
        mu = jnp.mean(z, axis=0, keepdims=True)
        var = jnp.mean((z - mu) ** 2, axis=0, keepdims=True)
        hn = (z - mu) * jax.lax.rsqrt(var + 1e-5) * gamma + beta
        h_ref[...] = jnp.maximum(hn, 0.0)

    @pl.when((l == 1) & (i == NB - 1))
    def _head():
        h = h_ref[...]
        hp = jnp.mean(h, axis=0, keepdims=True)
        node_ids = jax.lax.broadcasted_iota(jnp.int32, (P, N), 1)
        oh = (node_ids == cand_ref[...]).astype(jnp.float32)
        cf = jnp.dot(oh, h, preferred_element_type=jnp.float32)
        concat = jnp.concatenate(
            [cf, jnp.broadcast_to(hp, (P, F))], axis=1)
        a = jnp.tanh(concat @ wa1_ref[...] + ba1_ref[...])
        s = a @ wa2_ref[...] + ba2_ref[...]
        s = jnp.where(maskf_ref[...] > 0.0, -1e30, s)
        e = jnp.exp(s - jnp.max(s))
        pi_ref[...] = e / jnp.sum(e)
        c = jnp.tanh(hp @ wc1_ref[...] + bc1_ref[...])
        v_ref[...] = c @ wc2_ref[...] + bc2_ref[...]


def kernel(adj, x, candidate, mask,
           W0_1, b0_1, W0_2, b0_2, gamma0, beta0,
           W1_1, b1_1, W1_2, b1_2, gamma1, beta1,
           Wa1, ba1, Wa2, ba2, Wc1, bc1, Wc2, bc2):
    f32 = jnp.float32
    cand_col = jnp.full((P, 1), -1, jnp.int32).at[:NJ, 0].set(candidate[0])
    maskf = jnp.ones((P, 1), f32).at[:NJ, 0].set(mask[0].astype(f32))

    row = lambda v: v.reshape(1, -1).astype(f32)
    full = lambda a: pl.BlockSpec(a.shape, lambda l, i: (0,) * a.ndim)

    operands = (adj, x, cand_col, maskf,
                W0_1, row(b0_1), W0_2, row(b0_2), row(gamma0), row(beta0),
                W1_1, row(b1_1), W1_2, row(b1_2), row(gamma1), row(beta1),
                Wa1, row(ba1), Wa2, row(ba2),
                Wc1, row(bc1), Wc2, row(bc2))
    in_specs = [pl.BlockSpec((R, N), lambda l, i: (i, 0))]
    in_specs += [full(a) for a in operands[1:]]

    pi_pad, v = pl.pallas_call(
        _fwd_kernel,
        grid=(2, NB),
        in_specs=in_specs,
        out_specs=[pl.BlockSpec((P, 1), lambda l, i: (0, 0)),
                   pl.BlockSpec((1, 1), lambda l, i: (0, 0))],
        out_shape=[jax.ShapeDtypeStruct((P, 1), f32),
                   jax.ShapeDtypeStruct((1, 1), f32)],
        scratch_shapes=[pltpu.VMEM((N, F), f32),
                        pltpu.VMEM((N, F), f32)],
    )(*operands)
    return pi_pad[:NJ, 0][None, :], v
